# Initial kernel scaffold; baseline (speedup 1.0000x reference)
import jax
import jax.numpy as jnp
from jax.experimental import pallas as pl


def kernel(coordinates_radii, features, grid):
    raise NotImplementedError("write your pallas kernel here")


# speedup vs baseline: 73.6824x; 73.6824x over previous
import functools

import jax
import jax.numpy as jnp
from jax import lax
from jax.experimental import pallas as pl
from jax.experimental.pallas import tpu as pltpu
from jax.experimental.pallas import tpu_sc as plsc

VOL = 32
KNN = 36
NB = 4
NA = 2048
A = NB * NA
G = VOL ** 3
W = 7
C = W * W * W
CP = 352
KP = 48
NSEG = NB * G


def _knn_body(x_ref, y_ref, z_ref, seg_ref, d_scr, off_scr):
    i = pl.program_id(0)
    batch = i // 2

    x = x_ref[...]
    y = y_ref[...]
    z = z_ref[...]

    bx = jnp.clip(jnp.round(x) - 3.0, 0.0, float(VOL - W))
    by = jnp.clip(jnp.round(y) - 3.0, 0.0, float(VOL - W))
    bz = jnp.clip(jnp.round(z) - 3.0, 0.0, float(VOL - W))

    axs = [(x - (bx + ox)) ** 2 for ox in range(W)]
    ays = [(y - (by + oy)) ** 2 for oy in range(W)]
    azs = [(z - (bz + oz)) ** 2 for oz in range(W)]

    base_flat = (bx.astype(jnp.int32) * (VOL * VOL)
                 + by.astype(jnp.int32) * VOL
                 + bz.astype(jnp.int32)
                 + batch * G)

    big_off = jnp.full((8, 128), 10 * G, jnp.int32)
    for ox in range(W):
        for oy in range(W):
            axy = axs[ox] + ays[oy]
            for oz in range(W):
                c = (ox * W + oy) * W + oz
                d_scr[c] = axy + azs[oz]
                off_scr[c] = jnp.full((8, 128), ox * (VOL * VOL) + oy * VOL + oz,
                                      jnp.int32)
    inf = jnp.full((8, 128), jnp.inf, jnp.float32)
    for c in range(C, CP):
        d_scr[c] = inf
        off_scr[c] = big_off

    seg_ref[...] = jnp.full((KP, 8, 128), -1, jnp.int32)

    nchunk = CP // 16

    def one_round(k, _):
        def min_chunk(j, m):
            blk = d_scr[pl.ds(j * 16, 16)]
            return jnp.minimum(m, jnp.min(blk, axis=0))

        m = lax.fori_loop(0, nchunk, min_chunk, inf)

        def sel_chunk(j, s):
            blk = d_scr[pl.ds(j * 16, 16)]
            offs = off_scr[pl.ds(j * 16, 16)]
            cand = jnp.where(blk == m[None], offs, big_off[None])
            return jnp.minimum(s, jnp.min(cand, axis=0))

        sel = lax.fori_loop(0, nchunk, sel_chunk, big_off)

        def rem_chunk(j, _):
            blk = d_scr[pl.ds(j * 16, 16)]
            offs = off_scr[pl.ds(j * 16, 16)]
            d_scr[pl.ds(j * 16, 16)] = jnp.where(offs == sel[None], inf[None], blk)
            return 0

        lax.fori_loop(0, nchunk, rem_chunk, 0)

        seg_ref[k] = base_flat + sel
        return 0

    lax.fori_loop(0, KNN, one_round, 0)


def _run_knn(x, y, z):
    return pl.pallas_call(
        _knn_body,
        grid=(8,),
        in_specs=[pl.BlockSpec((8, 128), lambda i: (i, 0))] * 3,
        out_specs=pl.BlockSpec((KP, 8, 128), lambda i: (0, i, 0)),
        out_shape=jax.ShapeDtypeStruct((KP, 64, 128), jnp.int32),
        scratch_shapes=[
            pltpu.VMEM((CP, 8, 128), jnp.float32),
            pltpu.VMEM((CP, 8, 128), jnp.int32),
        ],
    )(x, y, z)


NT = 32
SEG_PER_TILE = NSEG // NT
PAIRS_PER_BATCH = NA * KP
CHUNK = 8192


def _sc_body(seg_hbm, f0_hbm, f1_hbm, f2_hbm, out_hbm,
             seg_v, f0_v, f1_v, f2_v, amax, amin, asum, acnt, outv):
    cid = lax.axis_index("c")
    sid = lax.axis_index("s")
    tid = sid * 2 + cid
    batch = tid // (NT // NB)
    seg_base = tid * SEG_PER_TILE
    pair_base = batch * PAIRS_PER_BATCH

    neg = jnp.full((16,), -3.0e38, jnp.float32)
    pos = jnp.full((16,), 3.0e38, jnp.float32)
    zero = jnp.zeros((16,), jnp.float32)
    one = jnp.ones((16,), jnp.float32)

    def init_i(i, _):
        sl = pl.ds(i * 16, 16)
        amax[sl] = neg
        amin[sl] = pos
        asum[sl] = zero
        acnt[sl] = zero
        return 0

    lax.fori_loop(0, SEG_PER_TILE // 16, init_i, 0)

    def chunk_loop(ci, _):
        off = pair_base + ci * CHUNK
        pltpu.sync_copy(seg_hbm.at[pl.ds(off, CHUNK)], seg_v)
        pltpu.sync_copy(f0_hbm.at[pl.ds(off, CHUNK)], f0_v)
        pltpu.sync_copy(f1_hbm.at[pl.ds(off, CHUNK)], f1_v)
        pltpu.sync_copy(f2_hbm.at[pl.ds(off, CHUNK)], f2_v)

        def vec_loop(i, _):
            sl = pl.ds(i * 16, 16)
            s = seg_v[sl]
            local = s - seg_base
            msk = (local >= 0) & (local < SEG_PER_TILE)
            lsafe = jnp.where(msk, local, 0)
            g0 = plsc.load_gather(amax, [lsafe], mask=msk)
            plsc.store_scatter(amax, [lsafe], jnp.maximum(g0, f0_v[sl]), mask=msk)
            g1 = plsc.load_gather(amin, [lsafe], mask=msk)
            plsc.store_scatter(amin, [lsafe], jnp.minimum(g1, f1_v[sl]), mask=msk)
            plsc.addupdate_scatter(asum, [lsafe], f2_v[sl], mask=msk)
            plsc.addupdate_scatter(acnt, [lsafe], one, mask=msk)
            return 0

        lax.fori_loop(0, CHUNK // 16, vec_loop, 0)
        return 0

    lax.fori_loop(0, PAIRS_PER_BATCH // CHUNK, chunk_loop, 0)

    def fin_max(i, _):
        sl = pl.ds(i * 16, 16)
        outv[sl] = jnp.where(acnt[sl] > 0.0, amax[sl], 0.0)
        return 0

    def fin_min(i, _):
        sl = pl.ds(i * 16, 16)
        outv[sl] = jnp.where(acnt[sl] > 0.0, amin[sl], 0.0)
        return 0

    def fin_mean(i, _):
        sl = pl.ds(i * 16, 16)
        cnt = acnt[sl]
        outv[sl] = jnp.where(cnt > 0.0, asum[sl] / jnp.maximum(cnt, 1.0), 0.0)
        return 0

    lax.fori_loop(0, SEG_PER_TILE // 16, fin_max, 0)
    pltpu.sync_copy(outv, out_hbm.at[pl.ds(seg_base, SEG_PER_TILE)])
    lax.fori_loop(0, SEG_PER_TILE // 16, fin_min, 0)
    pltpu.sync_copy(outv, out_hbm.at[pl.ds(NSEG + seg_base, SEG_PER_TILE)])
    lax.fori_loop(0, SEG_PER_TILE // 16, fin_mean, 0)
    pltpu.sync_copy(outv, out_hbm.at[pl.ds(2 * NSEG + seg_base, SEG_PER_TILE)])


def _run_scatter(seg_pairs, f0p, f1p, f2p):
    mesh = plsc.VectorSubcoreMesh(core_axis_name="c", subcore_axis_name="s")
    run = functools.partial(
        pl.kernel,
        mesh=mesh,
        compiler_params=pltpu.CompilerParams(needs_layout_passes=False),
        out_type=jax.ShapeDtypeStruct((3 * NSEG,), jnp.float32),
        scratch_types=[
            pltpu.VMEM((CHUNK,), jnp.int32),
            pltpu.VMEM((CHUNK,), jnp.float32),
            pltpu.VMEM((CHUNK,), jnp.float32),
            pltpu.VMEM((CHUNK,), jnp.float32),
            pltpu.VMEM((SEG_PER_TILE,), jnp.float32),
            pltpu.VMEM((SEG_PER_TILE,), jnp.float32),
            pltpu.VMEM((SEG_PER_TILE,), jnp.float32),
            pltpu.VMEM((SEG_PER_TILE,), jnp.float32),
            pltpu.VMEM((SEG_PER_TILE,), jnp.float32),
        ],
    )(_sc_body)
    return run(seg_pairs, f0p, f1p, f2p)


def kernel(coordinates_radii, features, grid):
    coords = coordinates_radii[..., :3].reshape(A, 3)
    x = coords[:, 0].reshape(64, 128)
    y = coords[:, 1].reshape(64, 128)
    z = coords[:, 2].reshape(64, 128)

    seg48 = _run_knn(x, y, z)
    seg_pairs = seg48.reshape(KP, A).T.reshape(A * KP)

    f = features.reshape(A, 3)
    f0p = jnp.broadcast_to(f[:, 0:1], (A, KP)).reshape(A * KP)
    f1p = jnp.broadcast_to(f[:, 1:2], (A, KP)).reshape(A * KP)
    f2p = jnp.broadcast_to(f[:, 2:3], (A, KP)).reshape(A * KP)

    out3 = _run_scatter(seg_pairs, f0p, f1p, f2p)
    return out3.reshape(3, NSEG).T


# speedup vs baseline: 89.6115x; 1.2162x over previous
import functools

import jax
import jax.numpy as jnp
from jax import lax
from jax.experimental import pallas as pl
from jax.experimental.pallas import tpu as pltpu
from jax.experimental.pallas import tpu_sc as plsc

VOL = 32
KNN = 36
NB = 4
NA = 2048
A = NB * NA
G = VOL ** 3
W = 7
C = W * W * W
CP = 352
KP = 48
NSEG = NB * G


def _knn_body(x_ref, y_ref, z_ref, seg_ref, d_scr, off_scr):
    x = x_ref[...]
    y = y_ref[...]
    z = z_ref[...]

    bx = jnp.clip(jnp.round(x) - 3.0, 0.0, float(VOL - W))
    by = jnp.clip(jnp.round(y) - 3.0, 0.0, float(VOL - W))
    bz = jnp.clip(jnp.round(z) - 3.0, 0.0, float(VOL - W))

    axs = [(x - (bx + ox)) ** 2 for ox in range(W)]
    ays = [(y - (by + oy)) ** 2 for oy in range(W)]
    azs = [(z - (bz + oz)) ** 2 for oz in range(W)]

    base_flat = (bx.astype(jnp.int32) * (VOL * VOL)
                 + by.astype(jnp.int32) * VOL
                 + bz.astype(jnp.int32))

    big_off = jnp.full((8, 128), 10 * G, jnp.int32)
    for ox in range(W):
        for oy in range(W):
            axy = axs[ox] + ays[oy]
            for oz in range(W):
                c = (ox * W + oy) * W + oz
                d_scr[c] = axy + azs[oz]
                off_scr[c] = jnp.full((8, 128), ox * (VOL * VOL) + oy * VOL + oz,
                                      jnp.int32)
    inf = jnp.full((8, 128), jnp.inf, jnp.float32)
    for c in range(C, CP):
        d_scr[c] = inf
        off_scr[c] = big_off

    seg_ref[...] = G + lax.broadcasted_iota(jnp.int32, (KP, 8, 128), 0)

    nchunk = CP // 16

    def one_round(k, _):
        def min_chunk(j, m):
            blk = d_scr[pl.ds(j * 16, 16)]
            return jnp.minimum(m, jnp.min(blk, axis=0))

        m = lax.fori_loop(0, nchunk, min_chunk, inf)

        def sel_chunk(j, s):
            blk = d_scr[pl.ds(j * 16, 16)]
            offs = off_scr[pl.ds(j * 16, 16)]
            cand = jnp.where(blk == m[None], offs, big_off[None])
            return jnp.minimum(s, jnp.min(cand, axis=0))

        sel = lax.fori_loop(0, nchunk, sel_chunk, big_off)

        def rem_chunk(j, _):
            blk = d_scr[pl.ds(j * 16, 16)]
            offs = off_scr[pl.ds(j * 16, 16)]
            d_scr[pl.ds(j * 16, 16)] = jnp.where(offs == sel[None], inf[None], blk)
            return 0

        lax.fori_loop(0, nchunk, rem_chunk, 0)

        seg_ref[k] = base_flat + sel
        return 0

    lax.fori_loop(0, KNN, one_round, 0)


def _run_knn(x, y, z):
    return pl.pallas_call(
        _knn_body,
        grid=(8,),
        in_specs=[pl.BlockSpec((8, 128), lambda i: (i, 0))] * 3,
        out_specs=pl.BlockSpec((KP, 8, 128), lambda i: (0, i, 0)),
        out_shape=jax.ShapeDtypeStruct((KP, 64, 128), jnp.int32),
        scratch_shapes=[
            pltpu.VMEM((CP, 8, 128), jnp.float32),
            pltpu.VMEM((CP, 8, 128), jnp.int32),
        ],
    )(x, y, z)


NT = 32
SEG_PER_TILE = NSEG // NT
PAIRS_PER_BATCH = NA * KP
CHUNK = 6144
ATOMS_PER_CHUNK = CHUNK // KP
NCH = PAIRS_PER_BATCH // CHUNK


def _sc_body(seg_hbm, ftab_hbm, out_hbm,
             seg_v, ftab_v, amax, amin, asum, acnt, outv, sem0, sem1):
    cid = lax.axis_index("c")
    sid = lax.axis_index("s")
    tid = sid * 2 + cid
    batch = tid // (NT // NB)
    r = tid % (NT // NB)
    vox_base = r * SEG_PER_TILE
    pair_base = batch * PAIRS_PER_BATCH

    neg = jnp.full((16,), -3.0e38, jnp.float32)
    pos = jnp.full((16,), 3.0e38, jnp.float32)
    zero = jnp.zeros((16,), jnp.float32)
    one = jnp.ones((16,), jnp.float32)

    def init_i(i, _):
        sl = pl.ds(i * 16, 16)
        amax[sl] = neg
        amin[sl] = pos
        asum[sl] = zero
        acnt[sl] = zero
        return 0

    lax.fori_loop(0, SEG_PER_TILE // 16, init_i, 0)

    pltpu.sync_copy(ftab_hbm, ftab_v)

    def start(ci, buf, sem):
        return pltpu.async_copy(
            seg_hbm.at[pl.ds(pair_base + ci * CHUNK, CHUNK)],
            seg_v.at[buf], sem)

    def process(ci, buf):
        abase = batch * NA + ci * ATOMS_PER_CHUNK

        def atom_loop(a, _):
            i0 = jnp.full((16,), abase + a, jnp.int32)
            f0 = plsc.load_gather(ftab_v, [i0])
            f1 = plsc.load_gather(ftab_v, [i0 + A])
            f2 = plsc.load_gather(ftab_v, [i0 + 2 * A])
            for j in range(3):
                s = seg_v[buf, pl.ds(a * KP + j * 16, 16)]
                local = s - vox_base
                msk = (local >= 0) & (local < SEG_PER_TILE)
                l = jnp.where(msk, local, 0)
                g0 = plsc.load_gather(amax, [l], mask=msk)
                plsc.store_scatter(amax, [l], jnp.maximum(g0, f0), mask=msk)
                g1 = plsc.load_gather(amin, [l], mask=msk)
                plsc.store_scatter(amin, [l], jnp.minimum(g1, f1), mask=msk)
                plsc.addupdate_scatter(asum, [l], f2, mask=msk)
                plsc.addupdate_scatter(acnt, [l], one, mask=msk)
            return 0

        lax.fori_loop(0, ATOMS_PER_CHUNK, atom_loop, 0)

    cps = [None, None]
    cps[0] = start(0, 0, sem0)
    for ci in range(NCH):
        buf = ci % 2
        cps[buf].wait()
        if ci + 1 < NCH:
            nbuf = (ci + 1) % 2
            cps[nbuf] = start(ci + 1, nbuf, sem1 if nbuf else sem0)
        process(ci, buf)

    out_base = batch * G + vox_base

    def fin_max(i, _):
        sl = pl.ds(i * 16, 16)
        outv[sl] = jnp.where(acnt[sl] > 0.0, amax[sl], 0.0)
        return 0

    def fin_min(i, _):
        sl = pl.ds(i * 16, 16)
        outv[sl] = jnp.where(acnt[sl] > 0.0, amin[sl], 0.0)
        return 0

    def fin_mean(i, _):
        sl = pl.ds(i * 16, 16)
        cnt = acnt[sl]
        outv[sl] = jnp.where(cnt > 0.0, asum[sl] / jnp.maximum(cnt, 1.0), 0.0)
        return 0

    nvec = SEG_PER_TILE // 16
    lax.fori_loop(0, nvec, fin_max, 0)
    pltpu.sync_copy(outv, out_hbm.at[pl.ds(out_base, SEG_PER_TILE)])
    lax.fori_loop(0, nvec, fin_min, 0)
    pltpu.sync_copy(outv, out_hbm.at[pl.ds(NSEG + out_base, SEG_PER_TILE)])
    lax.fori_loop(0, nvec, fin_mean, 0)
    pltpu.sync_copy(outv, out_hbm.at[pl.ds(2 * NSEG + out_base, SEG_PER_TILE)])


def _run_scatter(seg_pairs, ftab):
    mesh = plsc.VectorSubcoreMesh(core_axis_name="c", subcore_axis_name="s")
    run = functools.partial(
        pl.kernel,
        mesh=mesh,
        compiler_params=pltpu.CompilerParams(needs_layout_passes=False),
        out_type=jax.ShapeDtypeStruct((3 * NSEG,), jnp.float32),
        scratch_types=[
            pltpu.VMEM((2, CHUNK), jnp.int32),
            pltpu.VMEM((3 * A,), jnp.float32),
            pltpu.VMEM((SEG_PER_TILE,), jnp.float32),
            pltpu.VMEM((SEG_PER_TILE,), jnp.float32),
            pltpu.VMEM((SEG_PER_TILE,), jnp.float32),
            pltpu.VMEM((SEG_PER_TILE,), jnp.float32),
            pltpu.VMEM((SEG_PER_TILE,), jnp.float32),
            pltpu.SemaphoreType.DMA,
            pltpu.SemaphoreType.DMA,
        ],
    )(_sc_body)
    return run(seg_pairs, ftab)


def kernel(coordinates_radii, features, grid):
    coords = coordinates_radii[..., :3].reshape(A, 3)
    x = coords[:, 0].reshape(64, 128)
    y = coords[:, 1].reshape(64, 128)
    z = coords[:, 2].reshape(64, 128)

    seg48 = _run_knn(x, y, z)
    seg_pairs = seg48.reshape(KP, A).T.reshape(A * KP)

    ftab = features.reshape(A, 3).T.reshape(3 * A)

    out3 = _run_scatter(seg_pairs, ftab)
    return out3.reshape(3, NSEG).T


# speedup vs baseline: 135.4300x; 1.5113x over previous
import functools

import jax
import jax.numpy as jnp
from jax import lax
from jax.experimental import pallas as pl
from jax.experimental.pallas import tpu as pltpu
from jax.experimental.pallas import tpu_sc as plsc

VOL = 32
KNN = 36
NB = 4
NA = 2048
A = NB * NA
G = VOL ** 3
W = 7
C = W * W * W
CP = 352
KP = 48
NSEG = NB * G


def _knn_body(x_ref, y_ref, z_ref, seg_ref, d_scr, off_scr):
    x = x_ref[...]
    y = y_ref[...]
    z = z_ref[...]

    bx = jnp.clip(jnp.round(x) - 3.0, 0.0, float(VOL - W))
    by = jnp.clip(jnp.round(y) - 3.0, 0.0, float(VOL - W))
    bz = jnp.clip(jnp.round(z) - 3.0, 0.0, float(VOL - W))

    axs = [(x - (bx + ox)) ** 2 for ox in range(W)]
    ays = [(y - (by + oy)) ** 2 for oy in range(W)]
    azs = [(z - (bz + oz)) ** 2 for oz in range(W)]

    base_flat = (bx.astype(jnp.int32) * (VOL * VOL)
                 + by.astype(jnp.int32) * VOL
                 + bz.astype(jnp.int32))

    big_off = jnp.full((8, 128), 10 * G, jnp.int32)
    for ox in range(W):
        for oy in range(W):
            axy = axs[ox] + ays[oy]
            for oz in range(W):
                c = (ox * W + oy) * W + oz
                d_scr[c] = axy + azs[oz]
                off_scr[c] = jnp.full((8, 128), ox * (VOL * VOL) + oy * VOL + oz,
                                      jnp.int32)
    inf = jnp.full((8, 128), jnp.inf, jnp.float32)
    for c in range(C, CP):
        d_scr[c] = inf
        off_scr[c] = big_off

    seg_ref[...] = G + lax.broadcasted_iota(jnp.int32, (KP, 8, 128), 0)

    nchunk = CP // 16

    def one_round(k, sel_prev):
        def fused_chunk(j, m):
            blk = d_scr[pl.ds(j * 16, 16)]
            offs = off_scr[pl.ds(j * 16, 16)]
            blk = jnp.where(offs == sel_prev[None], inf[None], blk)
            d_scr[pl.ds(j * 16, 16)] = blk
            return jnp.minimum(m, jnp.min(blk, axis=0))

        m = lax.fori_loop(0, nchunk, fused_chunk, inf, unroll=nchunk)

        def sel_chunk(j, s):
            blk = d_scr[pl.ds(j * 16, 16)]
            offs = off_scr[pl.ds(j * 16, 16)]
            cand = jnp.where(blk == m[None], offs, big_off[None])
            return jnp.minimum(s, jnp.min(cand, axis=0))

        sel = lax.fori_loop(0, nchunk, sel_chunk, big_off, unroll=nchunk)

        seg_ref[k] = base_flat + sel
        return sel

    lax.fori_loop(0, KNN, one_round, big_off)


def _run_knn(x, y, z):
    return pl.pallas_call(
        _knn_body,
        grid=(8,),
        in_specs=[pl.BlockSpec((8, 128), lambda i: (i, 0))] * 3,
        out_specs=pl.BlockSpec((KP, 8, 128), lambda i: (0, i, 0)),
        out_shape=jax.ShapeDtypeStruct((KP, 64, 128), jnp.int32),
        scratch_shapes=[
            pltpu.VMEM((CP, 8, 128), jnp.float32),
            pltpu.VMEM((CP, 8, 128), jnp.int32),
        ],
    )(x, y, z)


NT = 32
SEG_PER_TILE = NSEG // NT
PAIRS_PER_BATCH = NA * KP
CHUNK = 6144
ATOMS_PER_CHUNK = CHUNK // KP
NCH = PAIRS_PER_BATCH // CHUNK


def _sc_body(seg_hbm, ftab_hbm, out_hbm,
             seg_v, ftab_v, amax, amin, asum, acnt, outv, sem0, sem1):
    cid = lax.axis_index("c")
    sid = lax.axis_index("s")
    tid = sid * 2 + cid
    batch = tid // (NT // NB)
    r = tid % (NT // NB)
    vox_base = r * SEG_PER_TILE
    pair_base = batch * PAIRS_PER_BATCH

    neg = jnp.full((16,), -3.0e38, jnp.float32)
    pos = jnp.full((16,), 3.0e38, jnp.float32)
    zero = jnp.zeros((16,), jnp.float32)
    one = jnp.ones((16,), jnp.float32)

    def init_i(i, _):
        sl = pl.ds(i * 16, 16)
        amax[sl] = neg
        amin[sl] = pos
        asum[sl] = zero
        acnt[sl] = zero
        return 0

    lax.fori_loop(0, SEG_PER_TILE // 16, init_i, 0)

    pltpu.sync_copy(ftab_hbm, ftab_v)

    def start(ci, buf, sem):
        return pltpu.async_copy(
            seg_hbm.at[pl.ds(pair_base + ci * CHUNK, CHUNK)],
            seg_v.at[buf], sem)

    def process(ci, buf):
        abase = batch * NA + ci * ATOMS_PER_CHUNK

        def atom_loop(a, _):
            i0 = jnp.full((16,), abase + a, jnp.int32)
            f0 = plsc.load_gather(ftab_v, [i0])
            f1 = plsc.load_gather(ftab_v, [i0 + A])
            f2 = plsc.load_gather(ftab_v, [i0 + 2 * A])
            for j in range(3):
                s = seg_v[buf, pl.ds(a * KP + j * 16, 16)]
                local = s - vox_base
                msk = (local >= 0) & (local < SEG_PER_TILE)
                l = jnp.where(msk, local, 0)
                g0 = plsc.load_gather(amax, [l], mask=msk)
                plsc.store_scatter(amax, [l], jnp.maximum(g0, f0), mask=msk)
                g1 = plsc.load_gather(amin, [l], mask=msk)
                plsc.store_scatter(amin, [l], jnp.minimum(g1, f1), mask=msk)
                plsc.addupdate_scatter(asum, [l], f2, mask=msk)
                plsc.addupdate_scatter(acnt, [l], one, mask=msk)
            return 0

        lax.fori_loop(0, ATOMS_PER_CHUNK, atom_loop, 0)

    cps = [None, None]
    cps[0] = start(0, 0, sem0)
    for ci in range(NCH):
        buf = ci % 2
        cps[buf].wait()
        if ci + 1 < NCH:
            nbuf = (ci + 1) % 2
            cps[nbuf] = start(ci + 1, nbuf, sem1 if nbuf else sem0)
        process(ci, buf)

    out_base = batch * G + vox_base

    def fin_max(i, _):
        sl = pl.ds(i * 16, 16)
        outv[sl] = jnp.where(acnt[sl] > 0.0, amax[sl], 0.0)
        return 0

    def fin_min(i, _):
        sl = pl.ds(i * 16, 16)
        outv[sl] = jnp.where(acnt[sl] > 0.0, amin[sl], 0.0)
        return 0

    def fin_mean(i, _):
        sl = pl.ds(i * 16, 16)
        cnt = acnt[sl]
        outv[sl] = jnp.where(cnt > 0.0, asum[sl] / jnp.maximum(cnt, 1.0), 0.0)
        return 0

    nvec = SEG_PER_TILE // 16
    lax.fori_loop(0, nvec, fin_max, 0)
    pltpu.sync_copy(outv, out_hbm.at[pl.ds(out_base, SEG_PER_TILE)])
    lax.fori_loop(0, nvec, fin_min, 0)
    pltpu.sync_copy(outv, out_hbm.at[pl.ds(NSEG + out_base, SEG_PER_TILE)])
    lax.fori_loop(0, nvec, fin_mean, 0)
    pltpu.sync_copy(outv, out_hbm.at[pl.ds(2 * NSEG + out_base, SEG_PER_TILE)])


def _run_scatter(seg_pairs, ftab):
    mesh = plsc.VectorSubcoreMesh(core_axis_name="c", subcore_axis_name="s")
    run = functools.partial(
        pl.kernel,
        mesh=mesh,
        compiler_params=pltpu.CompilerParams(needs_layout_passes=False),
        out_type=jax.ShapeDtypeStruct((3 * NSEG,), jnp.float32),
        scratch_types=[
            pltpu.VMEM((2, CHUNK), jnp.int32),
            pltpu.VMEM((3 * A,), jnp.float32),
            pltpu.VMEM((SEG_PER_TILE,), jnp.float32),
            pltpu.VMEM((SEG_PER_TILE,), jnp.float32),
            pltpu.VMEM((SEG_PER_TILE,), jnp.float32),
            pltpu.VMEM((SEG_PER_TILE,), jnp.float32),
            pltpu.VMEM((SEG_PER_TILE,), jnp.float32),
            pltpu.SemaphoreType.DMA,
            pltpu.SemaphoreType.DMA,
        ],
    )(_sc_body)
    return run(seg_pairs, ftab)


def kernel(coordinates_radii, features, grid):
    coords = coordinates_radii[..., :3].reshape(A, 3)
    x = coords[:, 0].reshape(64, 128)
    y = coords[:, 1].reshape(64, 128)
    z = coords[:, 2].reshape(64, 128)

    seg48 = _run_knn(x, y, z)
    seg_pairs = seg48.reshape(KP, A).T.reshape(A * KP)

    ftab = features.reshape(A, 3).T.reshape(3 * A)

    out3 = _run_scatter(seg_pairs, ftab)
    return out3.reshape(3, NSEG).T


# speedup vs baseline: 180.2264x; 1.3308x over previous
import functools

import jax
import jax.numpy as jnp
from jax import lax
from jax.experimental import pallas as pl
from jax.experimental.pallas import tpu as pltpu
from jax.experimental.pallas import tpu_sc as plsc

VOL = 32
KNN = 36
NB = 4
NA = 2048
A = NB * NA
G = VOL ** 3
W = 7
C = W * W * W
CP = 352
KP = 48
NSEG = NB * G


def _knn_body(x_ref, y_ref, z_ref, seg_ref, d_scr, off_scr):
    x = x_ref[...]
    y = y_ref[...]
    z = z_ref[...]

    bx = jnp.clip(jnp.round(x) - 3.0, 0.0, float(VOL - W))
    by = jnp.clip(jnp.round(y) - 3.0, 0.0, float(VOL - W))
    bz = jnp.clip(jnp.round(z) - 3.0, 0.0, float(VOL - W))

    axs = [(x - (bx + ox)) ** 2 for ox in range(W)]
    ays = [(y - (by + oy)) ** 2 for oy in range(W)]
    azs = [(z - (bz + oz)) ** 2 for oz in range(W)]

    base_flat = (bx.astype(jnp.int32) * (VOL * VOL)
                 + by.astype(jnp.int32) * VOL
                 + bz.astype(jnp.int32))

    big_off = jnp.full((8, 128), 10 * G, jnp.int32)
    for ox in range(W):
        for oy in range(W):
            axy = axs[ox] + ays[oy]
            for oz in range(W):
                c = (ox * W + oy) * W + oz
                d_scr[c] = axy + azs[oz]
                off_scr[c] = jnp.full((8, 128), ox * (VOL * VOL) + oy * VOL + oz,
                                      jnp.int32)
    inf = jnp.full((8, 128), jnp.inf, jnp.float32)
    for c in range(C, CP):
        d_scr[c] = inf
        off_scr[c] = big_off

    seg_ref[...] = G + lax.broadcasted_iota(jnp.int32, (KP, 8, 128), 0)

    nchunk = CP // 16

    def one_round(k, sel_prev):
        def fused_chunk(j, m):
            blk = d_scr[pl.ds(j * 16, 16)]
            offs = off_scr[pl.ds(j * 16, 16)]
            blk = jnp.where(offs == sel_prev[None], inf[None], blk)
            d_scr[pl.ds(j * 16, 16)] = blk
            return jnp.minimum(m, jnp.min(blk, axis=0))

        m = lax.fori_loop(0, nchunk, fused_chunk, inf, unroll=nchunk)

        def sel_chunk(j, s):
            blk = d_scr[pl.ds(j * 16, 16)]
            offs = off_scr[pl.ds(j * 16, 16)]
            cand = jnp.where(blk == m[None], offs, big_off[None])
            return jnp.minimum(s, jnp.min(cand, axis=0))

        sel = lax.fori_loop(0, nchunk, sel_chunk, big_off, unroll=nchunk)

        seg_ref[k] = base_flat + sel
        return sel

    lax.fori_loop(0, KNN, one_round, big_off)


def _run_knn(x, y, z):
    return pl.pallas_call(
        _knn_body,
        grid=(8,),
        in_specs=[pl.BlockSpec((8, 128), lambda i: (i, 0))] * 3,
        out_specs=pl.BlockSpec((KP, 8, 128), lambda i: (0, i, 0)),
        out_shape=jax.ShapeDtypeStruct((KP, 64, 128), jnp.int32),
        scratch_shapes=[
            pltpu.VMEM((CP, 8, 128), jnp.float32),
            pltpu.VMEM((CP, 8, 128), jnp.int32),
        ],
    )(x, y, z)


NT = 32
SEG_PER_TILE = NSEG // NT
PAIRS_PER_BATCH = NA * KP
QUARTER = PAIRS_PER_BATCH // 4
CHUNK = 6144
ATOMS_PER_CHUNK = CHUNK // KP
NCH = QUARTER // CHUNK
ACC_N = G + 64


def _sc_accum_body(seg_hbm, ftab_hbm, part_hbm,
                   seg_v, ftab_v, acc_a, acc_b, sem0, sem1):
    cid = lax.axis_index("c")
    sid = lax.axis_index("s")
    tid = sid * 2 + cid
    batch = tid // 8
    e = tid % 8
    q = e % 4
    is_maxmin = e < 4

    pair_base = batch * PAIRS_PER_BATCH + q * QUARTER

    neg = jnp.full((16,), -3.0e38, jnp.float32)
    pos = jnp.full((16,), 3.0e38, jnp.float32)
    zero = jnp.zeros((16,), jnp.float32)
    one = jnp.ones((16,), jnp.float32)

    init_a = jnp.where(is_maxmin, neg, zero)
    init_b = jnp.where(is_maxmin, pos, zero)

    def init_i(i, _):
        sl = pl.ds(i * 16, 16)
        acc_a[sl] = init_a
        acc_b[sl] = init_b
        return 0

    lax.fori_loop(0, ACC_N // 16, init_i, 0)

    @pl.when(is_maxmin)
    def _():
        pltpu.sync_copy(ftab_hbm.at[pl.ds(0, 2 * A)], ftab_v)

    @pl.when(jnp.logical_not(is_maxmin))
    def _():
        pltpu.sync_copy(ftab_hbm.at[pl.ds(2 * A, A)], ftab_v.at[pl.ds(0, A)])

    def start(ci, buf, sem):
        return pltpu.async_copy(
            seg_hbm.at[pl.ds(pair_base + ci * CHUNK, CHUNK)],
            seg_v.at[buf], sem)

    def process(ci, buf):
        abase = batch * NA + (q * QUARTER + ci * CHUNK) // KP

        @pl.when(is_maxmin)
        def _():
            def atom_loop(a, _):
                i0 = jnp.full((16,), abase + a, jnp.int32)
                f0 = plsc.load_gather(ftab_v, [i0])
                f1 = plsc.load_gather(ftab_v, [i0 + A])
                for j in range(3):
                    s = seg_v[buf, pl.ds(a * KP + j * 16, 16)]
                    g0 = plsc.load_gather(acc_a, [s])
                    plsc.store_scatter(acc_a, [s], jnp.maximum(g0, f0))
                    g1 = plsc.load_gather(acc_b, [s])
                    plsc.store_scatter(acc_b, [s], jnp.minimum(g1, f1))
                return 0

            lax.fori_loop(0, ATOMS_PER_CHUNK, atom_loop, 0)

        @pl.when(jnp.logical_not(is_maxmin))
        def _():
            def atom_loop(a, _):
                i0 = jnp.full((16,), abase + a, jnp.int32)
                f2 = plsc.load_gather(ftab_v, [i0])
                for j in range(3):
                    s = seg_v[buf, pl.ds(a * KP + j * 16, 16)]
                    plsc.addupdate_scatter(acc_a, [s], f2)
                    plsc.addupdate_scatter(acc_b, [s], one)
                return 0

            lax.fori_loop(0, ATOMS_PER_CHUNK, atom_loop, 0)

    cps = [None, None]
    cps[0] = start(0, 0, sem0)
    for ci in range(NCH):
        buf = ci % 2
        cps[buf].wait()
        if ci + 1 < NCH:
            nbuf = (ci + 1) % 2
            cps[nbuf] = start(ci + 1, nbuf, sem1 if nbuf else sem0)
        process(ci, buf)

    pltpu.sync_copy(acc_a.at[pl.ds(0, G)],
                    part_hbm.at[pl.ds((tid * 2) * G, G)])
    pltpu.sync_copy(acc_b.at[pl.ds(0, G)],
                    part_hbm.at[pl.ds((tid * 2 + 1) * G, G)])


def _sc_merge_body(part_hbm, out_hbm, t0, t1, t2, t3, cntv, outv,
                   sem0, sem1, sem2, sem3):
    cid = lax.axis_index("c")
    sid = lax.axis_index("s")
    tid = sid * 2 + cid
    batch = tid // 8
    e = tid % 8
    voff = e * SEG_PER_TILE
    nvec = SEG_PER_TILE // 16
    sems = [sem0, sem1, sem2, sem3]
    bufs = [t0, t1, t2, t3]

    def fetch4(src_tile0, a):
        cps = []
        for j in range(4):
            off = ((src_tile0 + j) * 2 + a) * G + voff
            cps.append(pltpu.async_copy(
                part_hbm.at[pl.ds(off, SEG_PER_TILE)], bufs[j], sems[j]))
        for cp in cps:
            cp.wait()

    mm0 = batch * 8
    sc0 = batch * 8 + 4

    fetch4(sc0, 1)

    def red_cnt(i, _):
        sl = pl.ds(i * 16, 16)
        cntv[sl] = (t0[sl] + t1[sl]) + (t2[sl] + t3[sl])
        return 0

    lax.fori_loop(0, nvec, red_cnt, 0)

    out_base = batch * G + voff

    fetch4(mm0, 0)

    def red_max(i, _):
        sl = pl.ds(i * 16, 16)
        m = jnp.maximum(jnp.maximum(t0[sl], t1[sl]),
                        jnp.maximum(t2[sl], t3[sl]))
        outv[sl] = jnp.where(cntv[sl] > 0.0, m, 0.0)
        return 0

    lax.fori_loop(0, nvec, red_max, 0)
    pltpu.sync_copy(outv, out_hbm.at[pl.ds(out_base, SEG_PER_TILE)])

    fetch4(mm0, 1)

    def red_min(i, _):
        sl = pl.ds(i * 16, 16)
        m = jnp.minimum(jnp.minimum(t0[sl], t1[sl]),
                        jnp.minimum(t2[sl], t3[sl]))
        outv[sl] = jnp.where(cntv[sl] > 0.0, m, 0.0)
        return 0

    lax.fori_loop(0, nvec, red_min, 0)
    pltpu.sync_copy(outv, out_hbm.at[pl.ds(NSEG + out_base, SEG_PER_TILE)])

    fetch4(sc0, 0)

    def red_mean(i, _):
        sl = pl.ds(i * 16, 16)
        s = (t0[sl] + t1[sl]) + (t2[sl] + t3[sl])
        cnt = cntv[sl]
        outv[sl] = jnp.where(cnt > 0.0, s / jnp.maximum(cnt, 1.0), 0.0)
        return 0

    lax.fori_loop(0, nvec, red_mean, 0)
    pltpu.sync_copy(outv, out_hbm.at[pl.ds(2 * NSEG + out_base, SEG_PER_TILE)])


def _run_scatter(seg_pairs, ftab):
    mesh = plsc.VectorSubcoreMesh(core_axis_name="c", subcore_axis_name="s")
    accum = functools.partial(
        pl.kernel,
        mesh=mesh,
        compiler_params=pltpu.CompilerParams(needs_layout_passes=False),
        out_type=jax.ShapeDtypeStruct((NT * 2 * G,), jnp.float32),
        scratch_types=[
            pltpu.VMEM((2, CHUNK), jnp.int32),
            pltpu.VMEM((2 * A,), jnp.float32),
            pltpu.VMEM((ACC_N,), jnp.float32),
            pltpu.VMEM((ACC_N,), jnp.float32),
            pltpu.SemaphoreType.DMA,
            pltpu.SemaphoreType.DMA,
        ],
    )(_sc_accum_body)
    part = accum(seg_pairs, ftab)
    merge = functools.partial(
        pl.kernel,
        mesh=mesh,
        compiler_params=pltpu.CompilerParams(needs_layout_passes=False),
        out_type=jax.ShapeDtypeStruct((3 * NSEG,), jnp.float32),
        scratch_types=[
            pltpu.VMEM((SEG_PER_TILE,), jnp.float32),
            pltpu.VMEM((SEG_PER_TILE,), jnp.float32),
            pltpu.VMEM((SEG_PER_TILE,), jnp.float32),
            pltpu.VMEM((SEG_PER_TILE,), jnp.float32),
            pltpu.VMEM((SEG_PER_TILE,), jnp.float32),
            pltpu.VMEM((SEG_PER_TILE,), jnp.float32),
            pltpu.SemaphoreType.DMA,
            pltpu.SemaphoreType.DMA,
            pltpu.SemaphoreType.DMA,
            pltpu.SemaphoreType.DMA,
        ],
    )(_sc_merge_body)
    return merge(part)


def kernel(coordinates_radii, features, grid):
    coords = coordinates_radii[..., :3].reshape(A, 3)
    x = coords[:, 0].reshape(64, 128)
    y = coords[:, 1].reshape(64, 128)
    z = coords[:, 2].reshape(64, 128)

    seg48 = _run_knn(x, y, z)
    seg_pairs = seg48.reshape(KP, A).T.reshape(A * KP)

    ftab = features.reshape(A, 3).T.reshape(3 * A)

    out3 = _run_scatter(seg_pairs, ftab)
    return out3.reshape(3, NSEG).T


# speedup vs baseline: 180.8034x; 1.0032x over previous
import functools

import jax
import jax.numpy as jnp
from jax import lax
from jax.experimental import pallas as pl
from jax.experimental.pallas import tpu as pltpu
from jax.experimental.pallas import tpu_sc as plsc

VOL = 32
KNN = 36
NB = 4
NA = 2048
A = NB * NA
G = VOL ** 3
W = 7
C = W * W * W
CP = 352
KP = 48
NSEG = NB * G


def _knn_body(x_ref, y_ref, z_ref, seg_ref, d_scr, off_scr):
    x = x_ref[...]
    y = y_ref[...]
    z = z_ref[...]

    bx = jnp.clip(jnp.round(x) - 3.0, 0.0, float(VOL - W))
    by = jnp.clip(jnp.round(y) - 3.0, 0.0, float(VOL - W))
    bz = jnp.clip(jnp.round(z) - 3.0, 0.0, float(VOL - W))

    axs = [(x - (bx + ox)) ** 2 for ox in range(W)]
    ays = [(y - (by + oy)) ** 2 for oy in range(W)]
    azs = [(z - (bz + oz)) ** 2 for oz in range(W)]

    base_flat = (bx.astype(jnp.int32) * (VOL * VOL)
                 + by.astype(jnp.int32) * VOL
                 + bz.astype(jnp.int32))

    big_off = jnp.full((8, 128), 10 * G, jnp.int32)
    for ox in range(W):
        for oy in range(W):
            axy = axs[ox] + ays[oy]
            for oz in range(W):
                c = (ox * W + oy) * W + oz
                d_scr[c] = axy + azs[oz]
                off_scr[c] = jnp.full((8, 128), ox * (VOL * VOL) + oy * VOL + oz,
                                      jnp.int32)
    inf = jnp.full((8, 128), jnp.inf, jnp.float32)
    for c in range(C, CP):
        d_scr[c] = inf
        off_scr[c] = big_off

    seg_ref[...] = G + lax.broadcasted_iota(jnp.int32, (KP, 8, 128), 0)

    nchunk = CP // 16

    def one_round(k, sel_prev):
        def fused_chunk(j, m):
            blk = d_scr[pl.ds(j * 16, 16)]
            offs = off_scr[pl.ds(j * 16, 16)]
            blk = jnp.where(offs == sel_prev[None], inf[None], blk)
            d_scr[pl.ds(j * 16, 16)] = blk
            return jnp.minimum(m, jnp.min(blk, axis=0))

        m = lax.fori_loop(0, nchunk, fused_chunk, inf, unroll=nchunk)

        def sel_chunk(j, s):
            blk = d_scr[pl.ds(j * 16, 16)]
            offs = off_scr[pl.ds(j * 16, 16)]
            cand = jnp.where(blk == m[None], offs, big_off[None])
            return jnp.minimum(s, jnp.min(cand, axis=0))

        sel = lax.fori_loop(0, nchunk, sel_chunk, big_off, unroll=nchunk)

        seg_ref[k] = base_flat + sel
        return sel

    lax.fori_loop(0, KNN, one_round, big_off)


def _run_knn(x, y, z):
    return pl.pallas_call(
        _knn_body,
        grid=(8,),
        in_specs=[pl.BlockSpec((8, 128), lambda i: (i, 0))] * 3,
        out_specs=pl.BlockSpec((KP, 8, 128), lambda i: (0, i, 0)),
        out_shape=jax.ShapeDtypeStruct((KP, 64, 128), jnp.int32),
        scratch_shapes=[
            pltpu.VMEM((CP, 8, 128), jnp.float32),
            pltpu.VMEM((CP, 8, 128), jnp.int32),
        ],
    )(x, y, z)


NT = 32
SEG_PER_TILE = NSEG // NT
PAIRS_PER_BATCH = NA * KP
QUARTER = PAIRS_PER_BATCH // 4
CHUNK = 6144
ATOMS_PER_CHUNK = CHUNK // KP
NCH = QUARTER // CHUNK
ACC_N = G + 64


def _sc_accum_body(seg_hbm, ftab_hbm, part_hbm,
                   seg_v, ftab_v, acc_a, acc_b, sem0, sem1):
    cid = lax.axis_index("c")
    sid = lax.axis_index("s")
    tid = sid * 2 + cid
    batch = tid // 8
    e = tid % 8
    q = e % 4
    is_maxmin = e < 4

    pair_base = batch * PAIRS_PER_BATCH + q * QUARTER

    neg = jnp.full((16,), -3.0e38, jnp.float32)
    pos = jnp.full((16,), 3.0e38, jnp.float32)
    zero = jnp.zeros((16,), jnp.float32)
    one = jnp.ones((16,), jnp.float32)

    init_a = jnp.where(is_maxmin, neg, zero)
    init_b = jnp.where(is_maxmin, pos, zero)

    def init_i(i, _):
        sl = pl.ds(i * 16, 16)
        acc_a[sl] = init_a
        acc_b[sl] = init_b
        return 0

    lax.fori_loop(0, ACC_N // 16, init_i, 0)

    @pl.when(is_maxmin)
    def _():
        pltpu.sync_copy(ftab_hbm.at[pl.ds(0, 2 * A)], ftab_v)

    @pl.when(jnp.logical_not(is_maxmin))
    def _():
        pltpu.sync_copy(ftab_hbm.at[pl.ds(2 * A, A)], ftab_v.at[pl.ds(0, A)])

    def start(ci, buf, sem):
        return pltpu.async_copy(
            seg_hbm.at[pl.ds(pair_base + ci * CHUNK, CHUNK)],
            seg_v.at[buf], sem)

    def process(ci, buf):
        abase = batch * NA + (q * QUARTER + ci * CHUNK) // KP

        @pl.when(is_maxmin)
        def _():
            def atom_loop(a, _):
                i0 = jnp.full((16,), abase + a, jnp.int32)
                f0 = plsc.load_gather(ftab_v, [i0])
                f1 = plsc.load_gather(ftab_v, [i0 + A])
                for j in range(3):
                    s = seg_v[buf, pl.ds(a * KP + j * 16, 16)]
                    g0 = plsc.load_gather(acc_a, [s])
                    plsc.store_scatter(acc_a, [s], jnp.maximum(g0, f0))
                    g1 = plsc.load_gather(acc_b, [s])
                    plsc.store_scatter(acc_b, [s], jnp.minimum(g1, f1))
                return 0

            lax.fori_loop(0, ATOMS_PER_CHUNK, atom_loop, 0, unroll=4)

        @pl.when(jnp.logical_not(is_maxmin))
        def _():
            def atom_loop(a, _):
                i0 = jnp.full((16,), abase + a, jnp.int32)
                f2 = plsc.load_gather(ftab_v, [i0])
                for j in range(3):
                    s = seg_v[buf, pl.ds(a * KP + j * 16, 16)]
                    plsc.addupdate_scatter(acc_a, [s], f2)
                    plsc.addupdate_scatter(acc_b, [s], one)
                return 0

            lax.fori_loop(0, ATOMS_PER_CHUNK, atom_loop, 0, unroll=4)

    cps = [None, None]
    cps[0] = start(0, 0, sem0)
    for ci in range(NCH):
        buf = ci % 2
        cps[buf].wait()
        if ci + 1 < NCH:
            nbuf = (ci + 1) % 2
            cps[nbuf] = start(ci + 1, nbuf, sem1 if nbuf else sem0)
        process(ci, buf)

    pltpu.sync_copy(acc_a.at[pl.ds(0, G)],
                    part_hbm.at[pl.ds((tid * 2) * G, G)])
    pltpu.sync_copy(acc_b.at[pl.ds(0, G)],
                    part_hbm.at[pl.ds((tid * 2 + 1) * G, G)])


def _sc_merge_body(part_hbm, out_hbm, t0, t1, t2, t3, cntv, outv,
                   sem0, sem1, sem2, sem3):
    cid = lax.axis_index("c")
    sid = lax.axis_index("s")
    tid = sid * 2 + cid
    batch = tid // 8
    e = tid % 8
    voff = e * SEG_PER_TILE
    nvec = SEG_PER_TILE // 16
    sems = [sem0, sem1, sem2, sem3]
    bufs = [t0, t1, t2, t3]

    def fetch4(src_tile0, a):
        cps = []
        for j in range(4):
            off = ((src_tile0 + j) * 2 + a) * G + voff
            cps.append(pltpu.async_copy(
                part_hbm.at[pl.ds(off, SEG_PER_TILE)], bufs[j], sems[j]))
        for cp in cps:
            cp.wait()

    mm0 = batch * 8
    sc0 = batch * 8 + 4

    fetch4(sc0, 1)

    def red_cnt(i, _):
        sl = pl.ds(i * 16, 16)
        cntv[sl] = (t0[sl] + t1[sl]) + (t2[sl] + t3[sl])
        return 0

    lax.fori_loop(0, nvec, red_cnt, 0)

    out_base = batch * G + voff

    fetch4(mm0, 0)

    def red_max(i, _):
        sl = pl.ds(i * 16, 16)
        m = jnp.maximum(jnp.maximum(t0[sl], t1[sl]),
                        jnp.maximum(t2[sl], t3[sl]))
        outv[sl] = jnp.where(cntv[sl] > 0.0, m, 0.0)
        return 0

    lax.fori_loop(0, nvec, red_max, 0)
    pltpu.sync_copy(outv, out_hbm.at[pl.ds(out_base, SEG_PER_TILE)])

    fetch4(mm0, 1)

    def red_min(i, _):
        sl = pl.ds(i * 16, 16)
        m = jnp.minimum(jnp.minimum(t0[sl], t1[sl]),
                        jnp.minimum(t2[sl], t3[sl]))
        outv[sl] = jnp.where(cntv[sl] > 0.0, m, 0.0)
        return 0

    lax.fori_loop(0, nvec, red_min, 0)
    pltpu.sync_copy(outv, out_hbm.at[pl.ds(NSEG + out_base, SEG_PER_TILE)])

    fetch4(sc0, 0)

    def red_mean(i, _):
        sl = pl.ds(i * 16, 16)
        s = (t0[sl] + t1[sl]) + (t2[sl] + t3[sl])
        cnt = cntv[sl]
        outv[sl] = jnp.where(cnt > 0.0, s / jnp.maximum(cnt, 1.0), 0.0)
        return 0

    lax.fori_loop(0, nvec, red_mean, 0)
    pltpu.sync_copy(outv, out_hbm.at[pl.ds(2 * NSEG + out_base, SEG_PER_TILE)])


def _run_scatter(seg_pairs, ftab):
    mesh = plsc.VectorSubcoreMesh(core_axis_name="c", subcore_axis_name="s")
    accum = functools.partial(
        pl.kernel,
        mesh=mesh,
        compiler_params=pltpu.CompilerParams(needs_layout_passes=False),
        out_type=jax.ShapeDtypeStruct((NT * 2 * G,), jnp.float32),
        scratch_types=[
            pltpu.VMEM((2, CHUNK), jnp.int32),
            pltpu.VMEM((2 * A,), jnp.float32),
            pltpu.VMEM((ACC_N,), jnp.float32),
            pltpu.VMEM((ACC_N,), jnp.float32),
            pltpu.SemaphoreType.DMA,
            pltpu.SemaphoreType.DMA,
        ],
    )(_sc_accum_body)
    part = accum(seg_pairs, ftab)
    merge = functools.partial(
        pl.kernel,
        mesh=mesh,
        compiler_params=pltpu.CompilerParams(needs_layout_passes=False),
        out_type=jax.ShapeDtypeStruct((3 * NSEG,), jnp.float32),
        scratch_types=[
            pltpu.VMEM((SEG_PER_TILE,), jnp.float32),
            pltpu.VMEM((SEG_PER_TILE,), jnp.float32),
            pltpu.VMEM((SEG_PER_TILE,), jnp.float32),
            pltpu.VMEM((SEG_PER_TILE,), jnp.float32),
            pltpu.VMEM((SEG_PER_TILE,), jnp.float32),
            pltpu.VMEM((SEG_PER_TILE,), jnp.float32),
            pltpu.SemaphoreType.DMA,
            pltpu.SemaphoreType.DMA,
            pltpu.SemaphoreType.DMA,
            pltpu.SemaphoreType.DMA,
        ],
    )(_sc_merge_body)
    return merge(part)


def kernel(coordinates_radii, features, grid):
    coords = coordinates_radii[..., :3].reshape(A, 3)
    x = coords[:, 0].reshape(64, 128)
    y = coords[:, 1].reshape(64, 128)
    z = coords[:, 2].reshape(64, 128)

    seg48 = _run_knn(x, y, z)
    seg_pairs = seg48.reshape(KP, A).T.reshape(A * KP)

    ftab = features.reshape(A, 3).T.reshape(3 * A)

    out3 = _run_scatter(seg_pairs, ftab)
    return out3.reshape(3, NSEG).T
